# Initial kernel scaffold; baseline (speedup 1.0000x reference)
import jax
import jax.numpy as jnp
from jax.experimental import pallas as pl


def kernel(H, Z, block_id, batch_id, perturb_mask, edges, edge_attr, global_mask, Wm0, bm0, Wu0, bu0, Wm1, bm1, Wu1, bu1, Wm2, bm2, Wu2, bu2):
    raise NotImplementedError("write your pallas kernel here")


# speedup vs baseline: 3.3420x; 3.3420x over previous
import functools

import jax
import jax.numpy as jnp
from jax import lax
from jax.experimental import pallas as pl
from jax.experimental.pallas import tpu as pltpu
from jax.experimental.pallas import tpu_sc as plsc

F32 = jnp.float32
NC = 2
NS = 16
NW = NC * NS
CH = 80


def _mesh():
    return plsc.VectorSubcoreMesh(
        core_axis_name="c", subcore_axis_name="s",
        num_cores=NC, num_subcores=NS)


def _sc_params():
    return pltpu.CompilerParams(needs_layout_passes=False)


def _wid():
    return lax.axis_index("s") * NC + lax.axis_index("c")


def _sc_scatter128(X, idx, NB):
    NA, HID = X.shape
    n_chunks = NA // CH
    iters = (n_chunks + NW - 1) // NW
    nb_chunks = NB // CH
    zit = (nb_chunks + NS - 1) // NS

    @functools.partial(
        pl.kernel,
        out_type=jax.ShapeDtypeStruct((NC, NB, HID), F32),
        mesh=_mesh(),
        compiler_params=_sc_params(),
        scratch_types=[
            pltpu.VMEM_SHARED((NB, HID), F32),
            pltpu.VMEM((CH, HID), F32),
            pltpu.VMEM((CH,), jnp.int32),
        ],
    )
    def scat_k(x_hbm, idx_hbm, out_hbm, acc, buf, ibuf):
        cid = lax.axis_index("c")
        sid = lax.axis_index("s")
        wid = sid * NC + cid

        zv = jnp.zeros((16,), F32)

        def zrow(r, carry):
            for q in range(HID // 16):
                buf[r, pl.ds(q * 16, 16)] = zv
            return carry

        lax.fori_loop(0, CH, zrow, 0)

        def zbody(j, carry):
            c = j * NS + sid

            @pl.when(c < nb_chunks)
            def _():
                pltpu.sync_copy(buf, acc.at[pl.ds(c * CH, CH)])
            return carry

        lax.fori_loop(0, zit, zbody, 0)
        plsc.subcore_barrier()

        def body(j, carry):
            c = j * NW + wid

            @pl.when(c < n_chunks)
            def _():
                base = c * CH
                pltpu.sync_copy(idx_hbm.at[pl.ds(base, CH)], ibuf)
                pltpu.sync_copy(x_hbm.at[pl.ds(base, CH)], buf)
                pltpu.sync_copy(buf, acc.at[ibuf], add=True)
            return carry

        lax.fori_loop(0, iters, body, 0)
        plsc.subcore_barrier()

        def obody(j, carry):
            c = j * NS + sid

            @pl.when(c < nb_chunks)
            def _():
                b = c * CH
                pltpu.sync_copy(acc.at[pl.ds(b, CH)], buf)
                pltpu.sync_copy(buf, out_hbm.at[cid, pl.ds(b, CH)])
            return carry

        lax.fori_loop(0, zit, obody, 0)

    return scat_k(X, idx)


def _sc_dist(zx, zy, zz, src, dst):
    NB = zx.shape[0]
    E = src.shape[0]
    n_chunks = E // CH
    iters = n_chunks // NW

    @functools.partial(
        pl.kernel,
        out_type=jax.ShapeDtypeStruct((E,), F32),
        mesh=_mesh(),
        compiler_params=_sc_params(),
        scratch_types=[
            pltpu.VMEM((NB,), F32),
            pltpu.VMEM((NB,), F32),
            pltpu.VMEM((NB,), F32),
            pltpu.VMEM((CH,), jnp.int32),
            pltpu.VMEM((CH,), jnp.int32),
            pltpu.VMEM((CH,), F32),
        ],
    )
    def dist_k(zx_hbm, zy_hbm, zz_hbm, src_hbm, dst_hbm, d2_hbm,
               xt, yt, zt, sbuf, dbuf, obuf):
        wid = _wid()
        pltpu.sync_copy(zx_hbm, xt)
        pltpu.sync_copy(zy_hbm, yt)
        pltpu.sync_copy(zz_hbm, zt)

        def body(j, carry):
            base = (j * NW + wid) * CH
            pltpu.sync_copy(src_hbm.at[pl.ds(base, CH)], sbuf)
            pltpu.sync_copy(dst_hbm.at[pl.ds(base, CH)], dbuf)
            for g in range(CH // 16):
                si = sbuf[pl.ds(g * 16, 16)]
                di = dbuf[pl.ds(g * 16, 16)]
                rx = plsc.load_gather(xt, [si]) - plsc.load_gather(xt, [di])
                ry = plsc.load_gather(yt, [si]) - plsc.load_gather(yt, [di])
                rz = plsc.load_gather(zt, [si]) - plsc.load_gather(zt, [di])
                obuf[pl.ds(g * 16, 16)] = rx * rx + ry * ry + rz * rz + 1e-8
            pltpu.sync_copy(obuf, d2_hbm.at[pl.ds(base, CH)])
            return carry

        lax.fori_loop(0, iters, body, 0)

    return dist_k(zx, zy, zz, src, dst)


def _sc_msg(A, Bt, C, src, dst):
    NB, HID = A.shape
    E = src.shape[0]
    n_chunks = E // CH
    iters = n_chunks // NW
    QV = HID // 16
    nb_chunks = NB // CH
    zit = (nb_chunks + NS - 1) // NS

    @functools.partial(
        pl.kernel,
        out_type=jax.ShapeDtypeStruct((NC, NB, HID), F32),
        mesh=_mesh(),
        compiler_params=_sc_params(),
        scratch_types=[
            pltpu.VMEM_SHARED((NB, HID), F32),
            pltpu.VMEM((CH,), jnp.int32),
            pltpu.VMEM((CH,), jnp.int32),
            pltpu.VMEM((CH, HID), F32),
            pltpu.VMEM((CH, HID), F32),
            pltpu.VMEM((CH, HID), F32),
            pltpu.SemaphoreType.DMA,
            pltpu.SemaphoreType.DMA,
            pltpu.SemaphoreType.DMA,
        ],
    )
    def msg_k(a_hbm, b_hbm, c_hbm, src_hbm, dst_hbm, agg_hbm,
              acc, sbuf, dbuf, ga, gb, cb, sem0, sem1, sem2):
        cid = lax.axis_index("c")
        sid = lax.axis_index("s")
        wid = sid * NC + cid

        zv = jnp.zeros((16,), F32)

        def zrow(r, carry):
            for q in range(QV):
                ga[r, pl.ds(q * 16, 16)] = zv
            return carry

        lax.fori_loop(0, CH, zrow, 0)

        def zbody(j, carry):
            c = j * NS + sid

            @pl.when(c < nb_chunks)
            def _():
                pltpu.sync_copy(ga, acc.at[pl.ds(c * CH, CH)])
            return carry

        lax.fori_loop(0, zit, zbody, 0)
        plsc.subcore_barrier()

        def body(j, carry):
            base = (j * NW + wid) * CH
            pltpu.sync_copy(src_hbm.at[pl.ds(base, CH)], sbuf)
            pltpu.sync_copy(dst_hbm.at[pl.ds(base, CH)], dbuf)
            cp0 = pltpu.async_copy(a_hbm.at[sbuf], ga, sem0)
            cp1 = pltpu.async_copy(b_hbm.at[dbuf], gb, sem1)
            cp2 = pltpu.async_copy(c_hbm.at[pl.ds(base, CH)], cb, sem2)
            cp0.wait()
            cp1.wait()
            cp2.wait()

            def row(r, rc):
                for q in range(QV):
                    x = (ga[r, pl.ds(q * 16, 16)]
                         + gb[r, pl.ds(q * 16, 16)]
                         + cb[r, pl.ds(q * 16, 16)])
                    ga[r, pl.ds(q * 16, 16)] = x / (1.0 + jnp.exp(-x))
                return rc

            lax.fori_loop(0, CH, row, 0)
            pltpu.sync_copy(ga, acc.at[dbuf], add=True)
            return carry

        lax.fori_loop(0, iters, body, 0)
        plsc.subcore_barrier()

        def obody(j, carry):
            c = j * NS + sid

            @pl.when(c < nb_chunks)
            def _():
                b = c * CH
                pltpu.sync_copy(acc.at[pl.ds(b, CH)], ga)
                pltpu.sync_copy(ga, agg_hbm.at[cid, pl.ds(b, CH)])
            return carry

        lax.fori_loop(0, zit, obody, 0)

    return msg_k(A, Bt, C, src, dst)


def _tc_combine(hsum, zsum, WmA, WmB):
    _, NB, HID = hsum.shape
    BK = 1000
    grid = NB // BK

    def body(hs_ref, zs_ref, wa_ref, wb_ref, hb_ref, zb_ref, a_ref, b_ref):
        hs = hs_ref[0] + hs_ref[1]
        zs = zs_ref[0] + zs_ref[1]
        cnt = zs[:, 3:4]
        inv = 1.0 / jnp.maximum(cnt, 1.0)
        hb = hs * inv
        hb_ref[...] = hb
        zb_ref[...] = zs[:, :16] * inv
        a_ref[...] = jnp.dot(hb, wa_ref[...], preferred_element_type=F32)
        b_ref[...] = jnp.dot(hb, wb_ref[...], preferred_element_type=F32)

    return pl.pallas_call(
        body,
        grid=(grid,),
        in_specs=[
            pl.BlockSpec((2, BK, HID), lambda i: (0, i, 0)),
            pl.BlockSpec((2, BK, HID), lambda i: (0, i, 0)),
            pl.BlockSpec((HID, HID), lambda i: (0, 0)),
            pl.BlockSpec((HID, HID), lambda i: (0, 0)),
        ],
        out_specs=[
            pl.BlockSpec((BK, HID), lambda i: (i, 0)),
            pl.BlockSpec((BK, 16), lambda i: (i, 0)),
            pl.BlockSpec((BK, HID), lambda i: (i, 0)),
            pl.BlockSpec((BK, HID), lambda i: (i, 0)),
        ],
        out_shape=[
            jax.ShapeDtypeStruct((NB, HID), F32),
            jax.ShapeDtypeStruct((NB, 16), F32),
            jax.ShapeDtypeStruct((NB, HID), F32),
            jax.ShapeDtypeStruct((NB, HID), F32),
        ],
    )(hsum, zsum, WmA, WmB)


def _tc_edge(edge_attr, d2, Wes, wds, bms):
    E, EDGE = edge_attr.shape
    HID = Wes[0].shape[1]
    BK = 2000
    grid = E // BK

    def body(ea_ref, d2_ref, we0, we1, we2, wd0, wd1, wd2, b0, b1, b2,
             c0_ref, c1_ref, c2_ref):
        ea = ea_ref[...]
        dist = jnp.sqrt(d2_ref[...])
        for w, wd, b, ref in ((we0, wd0, b0, c0_ref),
                              (we1, wd1, b1, c1_ref),
                              (we2, wd2, b2, c2_ref)):
            ref[...] = (jnp.dot(ea, w[...], preferred_element_type=F32)
                        + dist * wd[...] + b[...])

    wspec = pl.BlockSpec((EDGE, HID), lambda i: (0, 0))
    rspec = pl.BlockSpec((1, HID), lambda i: (0, 0))
    espec = pl.BlockSpec((BK, HID), lambda i: (i, 0))
    return pl.pallas_call(
        body,
        grid=(grid,),
        in_specs=[pl.BlockSpec((BK, EDGE), lambda i: (i, 0)),
                  pl.BlockSpec((BK, 1), lambda i: (i, 0)),
                  wspec, wspec, wspec, rspec, rspec, rspec,
                  rspec, rspec, rspec],
        out_specs=[espec, espec, espec],
        out_shape=[jax.ShapeDtypeStruct((E, HID), F32)] * 3,
    )(edge_attr, d2, Wes[0], Wes[1], Wes[2], wds[0], wds[1], wds[2],
      bms[0], bms[1], bms[2])


def _tc_update(h, aggp, WuH, WuA, bu, WmA, WmB):
    NB, HID = h.shape
    BK = 1000
    grid = NB // BK
    last = WmA is None

    if last:
        ins = (h, aggp, WuH, WuA, bu)
        n_out = 1
    else:
        ins = (h, aggp, WuH, WuA, bu, WmA, WmB)
        n_out = 3

    wspec = pl.BlockSpec((HID, HID), lambda i: (0, 0))
    hspec = pl.BlockSpec((BK, HID), lambda i: (i, 0))

    def body2(*refs):
        if last:
            h_ref, ag_ref, wh_ref, wa_ref, bu_ref, hn_ref = refs
        else:
            (h_ref, ag_ref, wh_ref, wa_ref, bu_ref, wma_ref, wmb_ref,
             hn_ref, a_ref, b_ref) = refs
        h0 = h_ref[...]
        agg = ag_ref[0] + ag_ref[1]
        pre = (jnp.dot(h0, wh_ref[...], preferred_element_type=F32)
               + jnp.dot(agg, wa_ref[...], preferred_element_type=F32)
               + bu_ref[...])
        hn = h0 + pre * (1.0 / (1.0 + jnp.exp(-pre)))
        hn_ref[...] = hn
        if not last:
            a_ref[...] = jnp.dot(hn, wma_ref[...], preferred_element_type=F32)
            b_ref[...] = jnp.dot(hn, wmb_ref[...], preferred_element_type=F32)

    in_specs = [hspec, pl.BlockSpec((2, BK, HID), lambda i: (0, i, 0)),
                wspec, wspec, pl.BlockSpec((1, HID), lambda i: (0, 0))]
    if not last:
        in_specs += [wspec, wspec]
    return pl.pallas_call(
        body2,
        grid=(grid,),
        in_specs=in_specs,
        out_specs=[hspec] * n_out,
        out_shape=[jax.ShapeDtypeStruct((NB, HID), F32)] * n_out,
    )(*ins)


def _tc_final(h, gmask, batch_row, B):
    NB, HID = h.shape
    BK = 1000
    grid = NB // BK

    def body(h_ref, m_ref, bid_ref, br_ref, gr_ref, gs_ref):
        i = pl.program_id(0)
        h0 = h_ref[...]
        nrm = jnp.sqrt(jnp.sum(h0 * h0, axis=1, keepdims=True))
        br = h0 / jnp.maximum(nrm, 1e-12)
        br_ref[...] = br
        masked = br * m_ref[...]
        bid = bid_ref[0]
        oh = (lax.broadcasted_iota(jnp.int32, (B, BK), 0)
              == jnp.broadcast_to(bid, (B, BK))).astype(F32)
        part = jnp.dot(oh, masked, preferred_element_type=F32)

        @pl.when(i == 0)
        def _():
            gs_ref[...] = jnp.zeros_like(gs_ref)

        gs_ref[...] += part

        @pl.when(i == grid - 1)
        def _():
            gs = gs_ref[...]
            gn = jnp.sqrt(jnp.sum(gs * gs, axis=1, keepdims=True))
            gr_ref[...] = gs / jnp.maximum(gn, 1e-12)

    return pl.pallas_call(
        body,
        grid=(grid,),
        in_specs=[
            pl.BlockSpec((BK, HID), lambda i: (i, 0)),
            pl.BlockSpec((BK, 1), lambda i: (i, 0)),
            pl.BlockSpec((1, 1, BK), lambda i: (i, 0, 0)),
        ],
        out_specs=[
            pl.BlockSpec((BK, HID), lambda i: (i, 0)),
            pl.BlockSpec((B, HID), lambda i: (0, 0)),
        ],
        out_shape=[
            jax.ShapeDtypeStruct((NB, HID), F32),
            jax.ShapeDtypeStruct((B, HID), F32),
        ],
        scratch_shapes=[pltpu.VMEM((B, HID), F32)],
    )(h, gmask, batch_row)


def kernel(H, Z, block_id, batch_id, perturb_mask, edges, edge_attr,
           global_mask, Wm0, bm0, Wu0, bu0, Wm1, bm1, Wu1, bu1,
           Wm2, bm2, Wu2, bu2):
    NA, HID = H.shape
    NB = global_mask.shape[0]
    E = edge_attr.shape[0]
    B = 32

    src = edges[0].astype(jnp.int32)
    dst = edges[1].astype(jnp.int32)
    bid = block_id.astype(jnp.int32)

    Zaug = jnp.concatenate(
        [Z.reshape(NA, 3), jnp.ones((NA, 1), F32),
         jnp.zeros((NA, HID - 4), F32)], axis=1)

    Wm = (Wm0, Wm1, Wm2)
    WmA = [w[:HID] for w in Wm]
    WmB = [w[HID:2 * HID] for w in Wm]
    Wes = [w[2 * HID:2 * HID + 16] for w in Wm]
    wds = [w[2 * HID + 16:] for w in Wm]
    bms = [b.reshape(1, HID) for b in (bm0, bm1, bm2)]
    Wu = (Wu0, Wu1, Wu2)
    WuH = [w[:HID] for w in Wu]
    WuA = [w[HID:] for w in Wu]
    bus = [b.reshape(1, HID) for b in (bu0, bu1, bu2)]

    hsum = _sc_scatter128(H, bid, NB)
    zsum = _sc_scatter128(Zaug, bid, NB)
    Hb, Zb16, A, Bt = _tc_combine(hsum, zsum, WmA[0], WmB[0])

    d2 = _sc_dist(Zb16[:, 0], Zb16[:, 1], Zb16[:, 2], src, dst)
    C0, C1, C2 = _tc_edge(edge_attr, d2.reshape(E, 1), Wes, wds, bms)
    Cs = (C0, C1, C2)

    h = Hb
    for l in range(3):
        aggp = _sc_msg(A, Bt, Cs[l], src, dst)
        if l < 2:
            h, A, Bt = _tc_update(h, aggp, WuH[l], WuA[l], bus[l],
                                  WmA[l + 1], WmB[l + 1])
        else:
            (h,) = _tc_update(h, aggp, WuH[l], WuA[l], bus[l], None, None)

    gm = global_mask.astype(F32).reshape(NB, 1)
    batch_row = batch_id.astype(jnp.int32).reshape(NB // 1000, 1, 1000)
    block_repr, graph_repr = _tc_final(h, gm, batch_row, B)
    return (Hb, block_repr, graph_repr, None)


# speedup vs baseline: 4.2322x; 1.2664x over previous
import functools

import jax
import jax.numpy as jnp
from jax import lax
from jax.experimental import pallas as pl
from jax.experimental.pallas import tpu as pltpu
from jax.experimental.pallas import tpu_sc as plsc

F32 = jnp.float32
NC = 2
NS = 16
NW = NC * NS
CH = 80


def _mesh():
    return plsc.VectorSubcoreMesh(
        core_axis_name="c", subcore_axis_name="s",
        num_cores=NC, num_subcores=NS)


def _sc_params():
    return pltpu.CompilerParams(needs_layout_passes=False)


def _wid():
    return lax.axis_index("s") * NC + lax.axis_index("c")


def _sc_scatter128(X, idx, NB):
    NA, HID = X.shape
    n_chunks = NA // CH
    iters = (n_chunks + NW - 1) // NW
    nb_chunks = NB // CH
    zit = (nb_chunks + NS - 1) // NS

    @functools.partial(
        pl.kernel,
        out_type=jax.ShapeDtypeStruct((NC, NB, HID), F32),
        mesh=_mesh(),
        compiler_params=_sc_params(),
        scratch_types=[
            pltpu.VMEM_SHARED((NB, HID), F32),
            pltpu.VMEM((CH, HID), F32),
            pltpu.VMEM((CH,), jnp.int32),
        ],
    )
    def scat_k(x_hbm, idx_hbm, out_hbm, acc, buf, ibuf):
        cid = lax.axis_index("c")
        sid = lax.axis_index("s")
        wid = sid * NC + cid

        zv = jnp.zeros((16,), F32)

        def zrow(r, carry):
            for q in range(HID // 16):
                buf[r, pl.ds(q * 16, 16)] = zv
            return carry

        lax.fori_loop(0, CH, zrow, 0)

        def zbody(j, carry):
            c = j * NS + sid

            @pl.when(c < nb_chunks)
            def _():
                pltpu.sync_copy(buf, acc.at[pl.ds(c * CH, CH)])
            return carry

        lax.fori_loop(0, zit, zbody, 0)
        plsc.subcore_barrier()

        def body(j, carry):
            c = j * NW + wid

            @pl.when(c < n_chunks)
            def _():
                base = c * CH
                pltpu.sync_copy(idx_hbm.at[pl.ds(base, CH)], ibuf)
                pltpu.sync_copy(x_hbm.at[pl.ds(base, CH)], buf)
                pltpu.sync_copy(buf, acc.at[ibuf], add=True)
            return carry

        lax.fori_loop(0, iters, body, 0)
        plsc.subcore_barrier()

        def obody(j, carry):
            c = j * NS + sid

            @pl.when(c < nb_chunks)
            def _():
                b = c * CH
                pltpu.sync_copy(acc.at[pl.ds(b, CH)], buf)
                pltpu.sync_copy(buf, out_hbm.at[cid, pl.ds(b, CH)])
            return carry

        lax.fori_loop(0, zit, obody, 0)

    return scat_k(X, idx)


def _sc_dist(zx, zy, zz, src, dst):
    NB = zx.shape[0]
    E = src.shape[0]
    n_chunks = E // CH
    iters = n_chunks // NW

    @functools.partial(
        pl.kernel,
        out_type=jax.ShapeDtypeStruct((E,), F32),
        mesh=_mesh(),
        compiler_params=_sc_params(),
        scratch_types=[
            pltpu.VMEM((NB,), F32),
            pltpu.VMEM((NB,), F32),
            pltpu.VMEM((NB,), F32),
            pltpu.VMEM((CH,), jnp.int32),
            pltpu.VMEM((CH,), jnp.int32),
            pltpu.VMEM((CH,), F32),
        ],
    )
    def dist_k(zx_hbm, zy_hbm, zz_hbm, src_hbm, dst_hbm, d2_hbm,
               xt, yt, zt, sbuf, dbuf, obuf):
        wid = _wid()
        pltpu.sync_copy(zx_hbm, xt)
        pltpu.sync_copy(zy_hbm, yt)
        pltpu.sync_copy(zz_hbm, zt)

        def body(j, carry):
            base = (j * NW + wid) * CH
            pltpu.sync_copy(src_hbm.at[pl.ds(base, CH)], sbuf)
            pltpu.sync_copy(dst_hbm.at[pl.ds(base, CH)], dbuf)
            for g in range(CH // 16):
                si = sbuf[pl.ds(g * 16, 16)]
                di = dbuf[pl.ds(g * 16, 16)]
                rx = plsc.load_gather(xt, [si]) - plsc.load_gather(xt, [di])
                ry = plsc.load_gather(yt, [si]) - plsc.load_gather(yt, [di])
                rz = plsc.load_gather(zt, [si]) - plsc.load_gather(zt, [di])
                obuf[pl.ds(g * 16, 16)] = rx * rx + ry * ry + rz * rz + 1e-8
            pltpu.sync_copy(obuf, d2_hbm.at[pl.ds(base, CH)])
            return carry

        lax.fori_loop(0, iters, body, 0)

    return dist_k(zx, zy, zz, src, dst)


def _sc_msg(A, Bt, C, src, dst):
    NB, HID = A.shape
    E = src.shape[0]
    CHM = 40
    n_chunks = E // CHM
    iters = n_chunks // NW
    QV = HID // 16
    nb_chunks = NB // CHM
    zit = (nb_chunks + NS - 1) // NS

    @functools.partial(
        pl.kernel,
        out_type=jax.ShapeDtypeStruct((NC, NB, HID), F32),
        mesh=_mesh(),
        compiler_params=_sc_params(),
        scratch_types=(
            [pltpu.VMEM_SHARED((NB, HID), F32)]
            + [pltpu.VMEM((CHM,), jnp.int32)] * 6
            + [pltpu.VMEM((CHM, HID), F32)] * 8
            + [pltpu.SemaphoreType.DMA] * 8
        ),
    )
    def msg_k(a_hbm, b_hbm, c_hbm, src_hbm, dst_hbm, agg_hbm,
              acc, sbuf0, sbuf1, dbuf0, dbuf1, dbufS0, dbufS1,
              ga0, ga1, gb0, gb1, cb0, cb1, mb0, mb1,
              semA0, semB0, semC0, semS0, semA1, semB1, semC1, semS1):
        sbuf = (sbuf0, sbuf1)
        dbuf = (dbuf0, dbuf1)
        dbufS = (dbufS0, dbufS1)
        ga = (ga0, ga1)
        gb = (gb0, gb1)
        cb = (cb0, cb1)
        mb = (mb0, mb1)
        semA = (semA0, semA1)
        semB = (semB0, semB1)
        semC = (semC0, semC1)
        semS = (semS0, semS1)
        cid = lax.axis_index("c")
        sid = lax.axis_index("s")
        wid = sid * NC + cid

        zv = jnp.zeros((16,), F32)

        def zrow(r, carry):
            for q in range(QV):
                mb[0][r, pl.ds(q * 16, 16)] = zv
            return carry

        lax.fori_loop(0, CHM, zrow, 0)

        def zbody(j, carry):
            c = j * NS + sid

            @pl.when(c < nb_chunks)
            def _():
                pltpu.sync_copy(mb[0], acc.at[pl.ds(c * CHM, CHM)])
            return carry

        lax.fori_loop(0, zit, zbody, 0)
        plsc.subcore_barrier()

        def issue(j, p):
            base = (j * NW + wid) * CHM
            pltpu.sync_copy(src_hbm.at[pl.ds(base, CHM)], sbuf[p])
            pltpu.sync_copy(dst_hbm.at[pl.ds(base, CHM)], dbuf[p])
            pltpu.async_copy(a_hbm.at[sbuf[p]], ga[p], semA[p])
            pltpu.async_copy(b_hbm.at[dbuf[p]], gb[p], semB[p])
            pltpu.async_copy(c_hbm.at[pl.ds(base, CHM)], cb[p], semC[p])

        issue(0, 0)
        issue(1, 1)

        def body(jj, carry):
            for p in range(2):
                j = jj * 2 + p

                @pl.when(j < iters)
                def _():
                    pltpu.make_async_copy(a_hbm.at[sbuf[p]], ga[p],
                                          semA[p]).wait()
                    pltpu.make_async_copy(b_hbm.at[dbuf[p]], gb[p],
                                          semB[p]).wait()
                    pltpu.make_async_copy(
                        c_hbm.at[pl.ds(0, CHM)], cb[p], semC[p]).wait()

                    def row(r, rc):
                        for q in range(QV):
                            x = (ga[p][r, pl.ds(q * 16, 16)]
                                 + gb[p][r, pl.ds(q * 16, 16)]
                                 + cb[p][r, pl.ds(q * 16, 16)])
                            mb[p][r, pl.ds(q * 16, 16)] = (
                                x / (1.0 + jnp.exp(-x)))
                        return rc

                    lax.fori_loop(0, CHM, row, 0)
                    pltpu.sync_copy(mb[p], acc.at[dbuf[p]], add=True)

                    @pl.when(j + 2 < iters)
                    def _():
                        issue(j + 2, p)
            return carry

        lax.fori_loop(0, (iters + 1) // 2, body, 0)
        plsc.subcore_barrier()

        def obody(j, carry):
            c = j * NS + sid

            @pl.when(c < nb_chunks)
            def _():
                b = c * CHM
                pltpu.sync_copy(acc.at[pl.ds(b, CHM)], ga0)
                pltpu.sync_copy(ga0, agg_hbm.at[cid, pl.ds(b, CHM)])
            return carry

        lax.fori_loop(0, zit, obody, 0)

    return msg_k(A, Bt, C, src, dst)


def _tc_combine(hsum, zsum, WmA, WmB):
    _, NB, HID = hsum.shape
    BK = 1000
    grid = NB // BK

    def body(hs_ref, zs_ref, wa_ref, wb_ref, hb_ref, zb_ref, a_ref, b_ref):
        hs = hs_ref[0] + hs_ref[1]
        zs = zs_ref[0] + zs_ref[1]
        cnt = zs[:, 3:4]
        inv = 1.0 / jnp.maximum(cnt, 1.0)
        hb = hs * inv
        hb_ref[...] = hb
        zb_ref[...] = zs[:, :16] * inv
        a_ref[...] = jnp.dot(hb, wa_ref[...], preferred_element_type=F32)
        b_ref[...] = jnp.dot(hb, wb_ref[...], preferred_element_type=F32)

    return pl.pallas_call(
        body,
        grid=(grid,),
        in_specs=[
            pl.BlockSpec((2, BK, HID), lambda i: (0, i, 0)),
            pl.BlockSpec((2, BK, HID), lambda i: (0, i, 0)),
            pl.BlockSpec((HID, HID), lambda i: (0, 0)),
            pl.BlockSpec((HID, HID), lambda i: (0, 0)),
        ],
        out_specs=[
            pl.BlockSpec((BK, HID), lambda i: (i, 0)),
            pl.BlockSpec((BK, 16), lambda i: (i, 0)),
            pl.BlockSpec((BK, HID), lambda i: (i, 0)),
            pl.BlockSpec((BK, HID), lambda i: (i, 0)),
        ],
        out_shape=[
            jax.ShapeDtypeStruct((NB, HID), F32),
            jax.ShapeDtypeStruct((NB, 16), F32),
            jax.ShapeDtypeStruct((NB, HID), F32),
            jax.ShapeDtypeStruct((NB, HID), F32),
        ],
    )(hsum, zsum, WmA, WmB)


def _tc_edge(edge_attr, d2, Wes, wds, bms):
    E, EDGE = edge_attr.shape
    HID = Wes[0].shape[1]
    BK = 2000
    grid = E // BK

    def body(ea_ref, d2_ref, we0, we1, we2, wd0, wd1, wd2, b0, b1, b2,
             c0_ref, c1_ref, c2_ref):
        ea = ea_ref[...]
        dist = jnp.sqrt(d2_ref[...])
        for w, wd, b, ref in ((we0, wd0, b0, c0_ref),
                              (we1, wd1, b1, c1_ref),
                              (we2, wd2, b2, c2_ref)):
            ref[...] = (jnp.dot(ea, w[...], preferred_element_type=F32)
                        + dist * wd[...] + b[...])

    wspec = pl.BlockSpec((EDGE, HID), lambda i: (0, 0))
    rspec = pl.BlockSpec((1, HID), lambda i: (0, 0))
    espec = pl.BlockSpec((BK, HID), lambda i: (i, 0))
    return pl.pallas_call(
        body,
        grid=(grid,),
        in_specs=[pl.BlockSpec((BK, EDGE), lambda i: (i, 0)),
                  pl.BlockSpec((BK, 1), lambda i: (i, 0)),
                  wspec, wspec, wspec, rspec, rspec, rspec,
                  rspec, rspec, rspec],
        out_specs=[espec, espec, espec],
        out_shape=[jax.ShapeDtypeStruct((E, HID), F32)] * 3,
    )(edge_attr, d2, Wes[0], Wes[1], Wes[2], wds[0], wds[1], wds[2],
      bms[0], bms[1], bms[2])


def _tc_update(h, aggp, WuH, WuA, bu, WmA, WmB):
    NB, HID = h.shape
    BK = 1000
    grid = NB // BK
    last = WmA is None

    if last:
        ins = (h, aggp, WuH, WuA, bu)
        n_out = 1
    else:
        ins = (h, aggp, WuH, WuA, bu, WmA, WmB)
        n_out = 3

    wspec = pl.BlockSpec((HID, HID), lambda i: (0, 0))
    hspec = pl.BlockSpec((BK, HID), lambda i: (i, 0))

    def body2(*refs):
        if last:
            h_ref, ag_ref, wh_ref, wa_ref, bu_ref, hn_ref = refs
        else:
            (h_ref, ag_ref, wh_ref, wa_ref, bu_ref, wma_ref, wmb_ref,
             hn_ref, a_ref, b_ref) = refs
        h0 = h_ref[...]
        agg = ag_ref[0] + ag_ref[1]
        pre = (jnp.dot(h0, wh_ref[...], preferred_element_type=F32)
               + jnp.dot(agg, wa_ref[...], preferred_element_type=F32)
               + bu_ref[...])
        hn = h0 + pre * (1.0 / (1.0 + jnp.exp(-pre)))
        hn_ref[...] = hn
        if not last:
            a_ref[...] = jnp.dot(hn, wma_ref[...], preferred_element_type=F32)
            b_ref[...] = jnp.dot(hn, wmb_ref[...], preferred_element_type=F32)

    in_specs = [hspec, pl.BlockSpec((2, BK, HID), lambda i: (0, i, 0)),
                wspec, wspec, pl.BlockSpec((1, HID), lambda i: (0, 0))]
    if not last:
        in_specs += [wspec, wspec]
    return pl.pallas_call(
        body2,
        grid=(grid,),
        in_specs=in_specs,
        out_specs=[hspec] * n_out,
        out_shape=[jax.ShapeDtypeStruct((NB, HID), F32)] * n_out,
    )(*ins)


def _tc_final(h, gmask, batch_row, B):
    NB, HID = h.shape
    BK = 1000
    grid = NB // BK

    def body(h_ref, m_ref, bid_ref, br_ref, gr_ref, gs_ref):
        i = pl.program_id(0)
        h0 = h_ref[...]
        nrm = jnp.sqrt(jnp.sum(h0 * h0, axis=1, keepdims=True))
        br = h0 / jnp.maximum(nrm, 1e-12)
        br_ref[...] = br
        masked = br * m_ref[...]
        bid = bid_ref[0]
        oh = (lax.broadcasted_iota(jnp.int32, (B, BK), 0)
              == jnp.broadcast_to(bid, (B, BK))).astype(F32)
        part = jnp.dot(oh, masked, preferred_element_type=F32)

        @pl.when(i == 0)
        def _():
            gs_ref[...] = jnp.zeros_like(gs_ref)

        gs_ref[...] += part

        @pl.when(i == grid - 1)
        def _():
            gs = gs_ref[...]
            gn = jnp.sqrt(jnp.sum(gs * gs, axis=1, keepdims=True))
            gr_ref[...] = gs / jnp.maximum(gn, 1e-12)

    return pl.pallas_call(
        body,
        grid=(grid,),
        in_specs=[
            pl.BlockSpec((BK, HID), lambda i: (i, 0)),
            pl.BlockSpec((BK, 1), lambda i: (i, 0)),
            pl.BlockSpec((1, 1, BK), lambda i: (i, 0, 0)),
        ],
        out_specs=[
            pl.BlockSpec((BK, HID), lambda i: (i, 0)),
            pl.BlockSpec((B, HID), lambda i: (0, 0)),
        ],
        out_shape=[
            jax.ShapeDtypeStruct((NB, HID), F32),
            jax.ShapeDtypeStruct((B, HID), F32),
        ],
        scratch_shapes=[pltpu.VMEM((B, HID), F32)],
    )(h, gmask, batch_row)


def kernel(H, Z, block_id, batch_id, perturb_mask, edges, edge_attr,
           global_mask, Wm0, bm0, Wu0, bu0, Wm1, bm1, Wu1, bu1,
           Wm2, bm2, Wu2, bu2):
    NA, HID = H.shape
    NB = global_mask.shape[0]
    E = edge_attr.shape[0]
    B = 32

    src = edges[0].astype(jnp.int32)
    dst = edges[1].astype(jnp.int32)
    bid = block_id.astype(jnp.int32)

    Zaug = jnp.concatenate(
        [Z.reshape(NA, 3), jnp.ones((NA, 1), F32),
         jnp.zeros((NA, HID - 4), F32)], axis=1)

    Wm = (Wm0, Wm1, Wm2)
    WmA = [w[:HID] for w in Wm]
    WmB = [w[HID:2 * HID] for w in Wm]
    Wes = [w[2 * HID:2 * HID + 16] for w in Wm]
    wds = [w[2 * HID + 16:] for w in Wm]
    bms = [b.reshape(1, HID) for b in (bm0, bm1, bm2)]
    Wu = (Wu0, Wu1, Wu2)
    WuH = [w[:HID] for w in Wu]
    WuA = [w[HID:] for w in Wu]
    bus = [b.reshape(1, HID) for b in (bu0, bu1, bu2)]

    hsum = _sc_scatter128(H, bid, NB)
    zsum = _sc_scatter128(Zaug, bid, NB)
    Hb, Zb16, A, Bt = _tc_combine(hsum, zsum, WmA[0], WmB[0])

    d2 = _sc_dist(Zb16[:, 0], Zb16[:, 1], Zb16[:, 2], src, dst)
    C0, C1, C2 = _tc_edge(edge_attr, d2.reshape(E, 1), Wes, wds, bms)
    Cs = (C0, C1, C2)

    h = Hb
    for l in range(3):
        aggp = _sc_msg(A, Bt, Cs[l], src, dst)
        if l < 2:
            h, A, Bt = _tc_update(h, aggp, WuH[l], WuA[l], bus[l],
                                  WmA[l + 1], WmB[l + 1])
        else:
            (h,) = _tc_update(h, aggp, WuH[l], WuA[l], bus[l], None, None)

    gm = global_mask.astype(F32).reshape(NB, 1)
    batch_row = batch_id.astype(jnp.int32).reshape(NB // 1000, 1, 1000)
    block_repr, graph_repr = _tc_final(h, gm, batch_row, B)
    return (Hb, block_repr, graph_repr, None)


# speedup vs baseline: 5.4799x; 1.2948x over previous
import functools

import jax
import jax.numpy as jnp
from jax import lax
from jax.experimental import pallas as pl
from jax.experimental.pallas import tpu as pltpu
from jax.experimental.pallas import tpu_sc as plsc

F32 = jnp.float32
NC = 2
NS = 16
NW = NC * NS
CH = 80


def _mesh():
    return plsc.VectorSubcoreMesh(
        core_axis_name="c", subcore_axis_name="s",
        num_cores=NC, num_subcores=NS)


def _sc_params():
    return pltpu.CompilerParams(needs_layout_passes=False)


def _wid():
    return lax.axis_index("s") * NC + lax.axis_index("c")


def _sc_scatter128(X, idx, NB):
    NA, HID = X.shape
    n_chunks = NA // CH
    iters = (n_chunks + NW - 1) // NW
    nb_chunks = NB // CH
    zit = (nb_chunks + NS - 1) // NS

    @functools.partial(
        pl.kernel,
        out_type=jax.ShapeDtypeStruct((NC, NB, HID), F32),
        mesh=_mesh(),
        compiler_params=_sc_params(),
        scratch_types=[
            pltpu.VMEM_SHARED((NB, HID), F32),
            pltpu.VMEM((CH, HID), F32),
            pltpu.VMEM((CH,), jnp.int32),
        ],
    )
    def scat_k(x_hbm, idx_hbm, out_hbm, acc, buf, ibuf):
        cid = lax.axis_index("c")
        sid = lax.axis_index("s")
        wid = sid * NC + cid

        zv = jnp.zeros((16,), F32)

        def zrow(r, carry):
            for q in range(HID // 16):
                buf[r, pl.ds(q * 16, 16)] = zv
            return carry

        lax.fori_loop(0, CH, zrow, 0)

        def zbody(j, carry):
            c = j * NS + sid

            @pl.when(c < nb_chunks)
            def _():
                pltpu.sync_copy(buf, acc.at[pl.ds(c * CH, CH)])
            return carry

        lax.fori_loop(0, zit, zbody, 0)
        plsc.subcore_barrier()

        def body(j, carry):
            c = j * NW + wid

            @pl.when(c < n_chunks)
            def _():
                base = c * CH
                pltpu.sync_copy(idx_hbm.at[pl.ds(base, CH)], ibuf)
                pltpu.sync_copy(x_hbm.at[pl.ds(base, CH)], buf)
                pltpu.sync_copy(buf, acc.at[ibuf], add=True)
            return carry

        lax.fori_loop(0, iters, body, 0)
        plsc.subcore_barrier()

        def obody(j, carry):
            c = j * NS + sid

            @pl.when(c < nb_chunks)
            def _():
                b = c * CH
                pltpu.sync_copy(acc.at[pl.ds(b, CH)], buf)
                pltpu.sync_copy(buf, out_hbm.at[cid, pl.ds(b, CH)])
            return carry

        lax.fori_loop(0, zit, obody, 0)

    return scat_k(X, idx)


def _sc_dist(zx, zy, zz, src, dst):
    NB = zx.shape[0]
    E = src.shape[0]
    n_chunks = E // CH
    iters = n_chunks // NW

    @functools.partial(
        pl.kernel,
        out_type=jax.ShapeDtypeStruct((E,), F32),
        mesh=_mesh(),
        compiler_params=_sc_params(),
        scratch_types=[
            pltpu.VMEM((NB,), F32),
            pltpu.VMEM((NB,), F32),
            pltpu.VMEM((NB,), F32),
            pltpu.VMEM((CH,), jnp.int32),
            pltpu.VMEM((CH,), jnp.int32),
            pltpu.VMEM((CH,), F32),
        ],
    )
    def dist_k(zx_hbm, zy_hbm, zz_hbm, src_hbm, dst_hbm, d2_hbm,
               xt, yt, zt, sbuf, dbuf, obuf):
        wid = _wid()
        pltpu.sync_copy(zx_hbm, xt)
        pltpu.sync_copy(zy_hbm, yt)
        pltpu.sync_copy(zz_hbm, zt)

        def body(j, carry):
            base = (j * NW + wid) * CH
            pltpu.sync_copy(src_hbm.at[pl.ds(base, CH)], sbuf)
            pltpu.sync_copy(dst_hbm.at[pl.ds(base, CH)], dbuf)
            for g in range(CH // 16):
                si = sbuf[pl.ds(g * 16, 16)]
                di = dbuf[pl.ds(g * 16, 16)]
                rx = plsc.load_gather(xt, [si]) - plsc.load_gather(xt, [di])
                ry = plsc.load_gather(yt, [si]) - plsc.load_gather(yt, [di])
                rz = plsc.load_gather(zt, [si]) - plsc.load_gather(zt, [di])
                obuf[pl.ds(g * 16, 16)] = rx * rx + ry * ry + rz * rz + 1e-8
            pltpu.sync_copy(obuf, d2_hbm.at[pl.ds(base, CH)])
            return carry

        lax.fori_loop(0, iters, body, 0)

    return dist_k(zx, zy, zz, src, dst)


def _sc_msg(A, Bt, C, src, dst):
    NB, HID = A.shape
    E = src.shape[0]
    CHM = 40
    n_chunks = E // CHM
    iters = n_chunks // NW
    QV = HID // 16
    nb_chunks = NB // CHM
    zit = (nb_chunks + NS - 1) // NS

    @functools.partial(
        pl.kernel,
        out_type=jax.ShapeDtypeStruct((NC, NB, HID), F32),
        mesh=_mesh(),
        compiler_params=_sc_params(),
        scratch_types=(
            [pltpu.VMEM_SHARED((NB, HID), F32)]
            + [pltpu.VMEM((CHM,), jnp.int32)] * 8
            + [pltpu.VMEM((CHM, HID), F32)] * 8
            + [pltpu.SemaphoreType.DMA] * 10
        ),
    )
    def msg_k(a_hbm, b_hbm, c_hbm, src_hbm, dst_hbm, agg_hbm,
              acc, sb0, sb1, sb2, sb3, db0, db1, db2, db3,
              ga0, ga1, gb0, gb1, cb0, cb1, mb0, mb1,
              semA0, semB0, semC0, semA1, semB1, semC1,
              semI0, semI1, semI2, semI3):
        sb = (sb0, sb1, sb2, sb3)
        db = (db0, db1, db2, db3)
        ga = (ga0, ga1)
        gb = (gb0, gb1)
        cb = (cb0, cb1)
        mb = (mb0, mb1)
        semA = (semA0, semA1)
        semB = (semB0, semB1)
        semC = (semC0, semC1)
        semI = (semI0, semI1, semI2, semI3)
        cid = lax.axis_index("c")
        sid = lax.axis_index("s")
        wid = sid * NC + cid

        zv = jnp.zeros((16,), F32)

        def zrow(r, carry):
            for q in range(QV):
                mb[0][r, pl.ds(q * 16, 16)] = zv
            return carry

        lax.fori_loop(0, CHM, zrow, 0)

        def zbody(j, carry):
            c = j * NS + sid

            @pl.when(c < nb_chunks)
            def _():
                pltpu.sync_copy(mb[0], acc.at[pl.ds(c * CHM, CHM)])
            return carry

        lax.fori_loop(0, zit, zbody, 0)
        plsc.subcore_barrier()

        def ebase(j):
            return (j * NW + wid) * CHM

        def issue_idx(j, q):
            base = ebase(j)
            pltpu.async_copy(src_hbm.at[pl.ds(base, CHM)], sb[q], semI[q])
            pltpu.async_copy(dst_hbm.at[pl.ds(base, CHM)], db[q], semI[q])

        def wait_idx(q):
            pltpu.make_async_copy(
                src_hbm.at[pl.ds(0, CHM)], sb[q], semI[q]).wait()
            pltpu.make_async_copy(
                dst_hbm.at[pl.ds(0, CHM)], db[q], semI[q]).wait()

        def issue_gath(j, q, p):
            base = ebase(j)
            pltpu.async_copy(a_hbm.at[sb[q]], ga[p], semA[p])
            pltpu.async_copy(b_hbm.at[db[q]], gb[p], semB[p])
            pltpu.async_copy(c_hbm.at[pl.ds(base, CHM)], cb[p], semC[p])

        def wait_gath(q, p):
            pltpu.make_async_copy(a_hbm.at[sb[q]], ga[p], semA[p]).wait()
            pltpu.make_async_copy(b_hbm.at[db[q]], gb[p], semB[p]).wait()
            pltpu.make_async_copy(
                c_hbm.at[pl.ds(0, CHM)], cb[p], semC[p]).wait()

        for t in range(4):
            issue_idx(t, t)
        for t in range(2):
            wait_idx(t)
            issue_gath(t, t, t)

        def body(jj, carry):
            for p4 in range(4):
                p = p4 % 2
                j4 = jj * 4 + p4

                @pl.when(j4 < iters)
                def _():
                    wait_gath(p4, p)

                    def row(r, rc):
                        for q in range(QV):
                            x = (ga[p][r, pl.ds(q * 16, 16)]
                                 + gb[p][r, pl.ds(q * 16, 16)]
                                 + cb[p][r, pl.ds(q * 16, 16)])
                            mb[p][r, pl.ds(q * 16, 16)] = (
                                x / (1.0 + jnp.exp(-x)))
                        return rc

                    lax.fori_loop(0, CHM, row, 0)
                    pltpu.sync_copy(mb[p], acc.at[db[p4]], add=True)

                    @pl.when(j4 + 2 < iters)
                    def _():
                        wait_idx((p4 + 2) % 4)
                        issue_gath(j4 + 2, (p4 + 2) % 4, p)

                    @pl.when(j4 + 4 < iters)
                    def _():
                        issue_idx(j4 + 4, p4)
            return carry

        lax.fori_loop(0, (iters + 3) // 4, body, 0)
        plsc.subcore_barrier()

        def obody(j, carry):
            c = j * NS + sid

            @pl.when(c < nb_chunks)
            def _():
                b = c * CHM
                pltpu.sync_copy(acc.at[pl.ds(b, CHM)], ga0)
                pltpu.sync_copy(ga0, agg_hbm.at[cid, pl.ds(b, CHM)])
            return carry

        lax.fori_loop(0, zit, obody, 0)

    return msg_k(A, Bt, C, src, dst)


def _tc_combine(hsum, zsum, WmA, WmB):
    _, NB, HID = hsum.shape
    BK = 1000
    grid = NB // BK

    def body(hs_ref, zs_ref, wa_ref, wb_ref, hb_ref, zb_ref, a_ref, b_ref):
        hs = hs_ref[0] + hs_ref[1]
        zs = zs_ref[0] + zs_ref[1]
        cnt = zs[:, 3:4]
        inv = 1.0 / jnp.maximum(cnt, 1.0)
        hb = hs * inv
        hb_ref[...] = hb
        zb_ref[...] = zs[:, :16] * inv
        a_ref[...] = jnp.dot(hb, wa_ref[...], preferred_element_type=F32)
        b_ref[...] = jnp.dot(hb, wb_ref[...], preferred_element_type=F32)

    return pl.pallas_call(
        body,
        grid=(grid,),
        in_specs=[
            pl.BlockSpec((2, BK, HID), lambda i: (0, i, 0)),
            pl.BlockSpec((2, BK, HID), lambda i: (0, i, 0)),
            pl.BlockSpec((HID, HID), lambda i: (0, 0)),
            pl.BlockSpec((HID, HID), lambda i: (0, 0)),
        ],
        out_specs=[
            pl.BlockSpec((BK, HID), lambda i: (i, 0)),
            pl.BlockSpec((BK, 16), lambda i: (i, 0)),
            pl.BlockSpec((BK, HID), lambda i: (i, 0)),
            pl.BlockSpec((BK, HID), lambda i: (i, 0)),
        ],
        out_shape=[
            jax.ShapeDtypeStruct((NB, HID), F32),
            jax.ShapeDtypeStruct((NB, 16), F32),
            jax.ShapeDtypeStruct((NB, HID), F32),
            jax.ShapeDtypeStruct((NB, HID), F32),
        ],
    )(hsum, zsum, WmA, WmB)


def _tc_edge(edge_attr, d2, Wes, wds, bms):
    E, EDGE = edge_attr.shape
    HID = Wes[0].shape[1]
    BK = 2000
    grid = E // BK

    def body(ea_ref, d2_ref, we0, we1, we2, wd0, wd1, wd2, b0, b1, b2,
             c0_ref, c1_ref, c2_ref):
        ea = ea_ref[...]
        dist = jnp.sqrt(d2_ref[...])
        for w, wd, b, ref in ((we0, wd0, b0, c0_ref),
                              (we1, wd1, b1, c1_ref),
                              (we2, wd2, b2, c2_ref)):
            ref[...] = (jnp.dot(ea, w[...], preferred_element_type=F32)
                        + dist * wd[...] + b[...])

    wspec = pl.BlockSpec((EDGE, HID), lambda i: (0, 0))
    rspec = pl.BlockSpec((1, HID), lambda i: (0, 0))
    espec = pl.BlockSpec((BK, HID), lambda i: (i, 0))
    return pl.pallas_call(
        body,
        grid=(grid,),
        in_specs=[pl.BlockSpec((BK, EDGE), lambda i: (i, 0)),
                  pl.BlockSpec((BK, 1), lambda i: (i, 0)),
                  wspec, wspec, wspec, rspec, rspec, rspec,
                  rspec, rspec, rspec],
        out_specs=[espec, espec, espec],
        out_shape=[jax.ShapeDtypeStruct((E, HID), F32)] * 3,
    )(edge_attr, d2, Wes[0], Wes[1], Wes[2], wds[0], wds[1], wds[2],
      bms[0], bms[1], bms[2])


def _tc_update(h, aggp, WuH, WuA, bu, WmA, WmB):
    NB, HID = h.shape
    BK = 1000
    grid = NB // BK
    last = WmA is None

    if last:
        ins = (h, aggp, WuH, WuA, bu)
        n_out = 1
    else:
        ins = (h, aggp, WuH, WuA, bu, WmA, WmB)
        n_out = 3

    wspec = pl.BlockSpec((HID, HID), lambda i: (0, 0))
    hspec = pl.BlockSpec((BK, HID), lambda i: (i, 0))

    def body2(*refs):
        if last:
            h_ref, ag_ref, wh_ref, wa_ref, bu_ref, hn_ref = refs
        else:
            (h_ref, ag_ref, wh_ref, wa_ref, bu_ref, wma_ref, wmb_ref,
             hn_ref, a_ref, b_ref) = refs
        h0 = h_ref[...]
        agg = ag_ref[0] + ag_ref[1]
        pre = (jnp.dot(h0, wh_ref[...], preferred_element_type=F32)
               + jnp.dot(agg, wa_ref[...], preferred_element_type=F32)
               + bu_ref[...])
        hn = h0 + pre * (1.0 / (1.0 + jnp.exp(-pre)))
        hn_ref[...] = hn
        if not last:
            a_ref[...] = jnp.dot(hn, wma_ref[...], preferred_element_type=F32)
            b_ref[...] = jnp.dot(hn, wmb_ref[...], preferred_element_type=F32)

    in_specs = [hspec, pl.BlockSpec((2, BK, HID), lambda i: (0, i, 0)),
                wspec, wspec, pl.BlockSpec((1, HID), lambda i: (0, 0))]
    if not last:
        in_specs += [wspec, wspec]
    return pl.pallas_call(
        body2,
        grid=(grid,),
        in_specs=in_specs,
        out_specs=[hspec] * n_out,
        out_shape=[jax.ShapeDtypeStruct((NB, HID), F32)] * n_out,
    )(*ins)


def _tc_final(h, gmask, batch_row, B):
    NB, HID = h.shape
    BK = 1000
    grid = NB // BK

    def body(h_ref, m_ref, bid_ref, br_ref, gr_ref, gs_ref):
        i = pl.program_id(0)
        h0 = h_ref[...]
        nrm = jnp.sqrt(jnp.sum(h0 * h0, axis=1, keepdims=True))
        br = h0 / jnp.maximum(nrm, 1e-12)
        br_ref[...] = br
        masked = br * m_ref[...]
        bid = bid_ref[0]
        oh = (lax.broadcasted_iota(jnp.int32, (B, BK), 0)
              == jnp.broadcast_to(bid, (B, BK))).astype(F32)
        part = jnp.dot(oh, masked, preferred_element_type=F32)

        @pl.when(i == 0)
        def _():
            gs_ref[...] = jnp.zeros_like(gs_ref)

        gs_ref[...] += part

        @pl.when(i == grid - 1)
        def _():
            gs = gs_ref[...]
            gn = jnp.sqrt(jnp.sum(gs * gs, axis=1, keepdims=True))
            gr_ref[...] = gs / jnp.maximum(gn, 1e-12)

    return pl.pallas_call(
        body,
        grid=(grid,),
        in_specs=[
            pl.BlockSpec((BK, HID), lambda i: (i, 0)),
            pl.BlockSpec((BK, 1), lambda i: (i, 0)),
            pl.BlockSpec((1, 1, BK), lambda i: (i, 0, 0)),
        ],
        out_specs=[
            pl.BlockSpec((BK, HID), lambda i: (i, 0)),
            pl.BlockSpec((B, HID), lambda i: (0, 0)),
        ],
        out_shape=[
            jax.ShapeDtypeStruct((NB, HID), F32),
            jax.ShapeDtypeStruct((B, HID), F32),
        ],
        scratch_shapes=[pltpu.VMEM((B, HID), F32)],
    )(h, gmask, batch_row)


def kernel(H, Z, block_id, batch_id, perturb_mask, edges, edge_attr,
           global_mask, Wm0, bm0, Wu0, bu0, Wm1, bm1, Wu1, bu1,
           Wm2, bm2, Wu2, bu2):
    NA, HID = H.shape
    NB = global_mask.shape[0]
    E = edge_attr.shape[0]
    B = 32

    src = edges[0].astype(jnp.int32)
    dst = edges[1].astype(jnp.int32)
    bid = block_id.astype(jnp.int32)

    Zaug = jnp.concatenate(
        [Z.reshape(NA, 3), jnp.ones((NA, 1), F32),
         jnp.zeros((NA, HID - 4), F32)], axis=1)

    Wm = (Wm0, Wm1, Wm2)
    WmA = [w[:HID] for w in Wm]
    WmB = [w[HID:2 * HID] for w in Wm]
    Wes = [w[2 * HID:2 * HID + 16] for w in Wm]
    wds = [w[2 * HID + 16:] for w in Wm]
    bms = [b.reshape(1, HID) for b in (bm0, bm1, bm2)]
    Wu = (Wu0, Wu1, Wu2)
    WuH = [w[:HID] for w in Wu]
    WuA = [w[HID:] for w in Wu]
    bus = [b.reshape(1, HID) for b in (bu0, bu1, bu2)]

    hsum = _sc_scatter128(H, bid, NB)
    zsum = _sc_scatter128(Zaug, bid, NB)
    Hb, Zb16, A, Bt = _tc_combine(hsum, zsum, WmA[0], WmB[0])

    d2 = _sc_dist(Zb16[:, 0], Zb16[:, 1], Zb16[:, 2], src, dst)
    C0, C1, C2 = _tc_edge(edge_attr, d2.reshape(E, 1), Wes, wds, bms)
    Cs = (C0, C1, C2)

    h = Hb
    for l in range(3):
        aggp = _sc_msg(A, Bt, Cs[l], src, dst)
        if l < 2:
            h, A, Bt = _tc_update(h, aggp, WuH[l], WuA[l], bus[l],
                                  WmA[l + 1], WmB[l + 1])
        else:
            (h,) = _tc_update(h, aggp, WuH[l], WuA[l], bus[l], None, None)

    gm = global_mask.astype(F32).reshape(NB, 1)
    batch_row = batch_id.astype(jnp.int32).reshape(NB // 1000, 1, 1000)
    block_repr, graph_repr = _tc_final(h, gm, batch_row, B)
    return (Hb, block_repr, graph_repr, None)


# speedup vs baseline: 5.5984x; 1.0216x over previous
import functools

import jax
import jax.numpy as jnp
from jax import lax
from jax.experimental import pallas as pl
from jax.experimental.pallas import tpu as pltpu
from jax.experimental.pallas import tpu_sc as plsc

F32 = jnp.float32
NC = 2
NS = 16
NW = NC * NS
CH = 80


def _mesh():
    return plsc.VectorSubcoreMesh(
        core_axis_name="c", subcore_axis_name="s",
        num_cores=NC, num_subcores=NS)


def _sc_params():
    return pltpu.CompilerParams(needs_layout_passes=False)


def _wid():
    return lax.axis_index("s") * NC + lax.axis_index("c")


def _sc_scatter128(X, idx, NB):
    NA, HID = X.shape
    n_chunks = NA // CH
    iters = (n_chunks + NW - 1) // NW
    nb_chunks = NB // CH
    zit = (nb_chunks + NS - 1) // NS

    @functools.partial(
        pl.kernel,
        out_type=jax.ShapeDtypeStruct((NC, NB, HID), F32),
        mesh=_mesh(),
        compiler_params=_sc_params(),
        scratch_types=(
            [pltpu.VMEM_SHARED((NB, HID), F32)]
            + [pltpu.VMEM((CH, HID), F32)] * 2
            + [pltpu.VMEM((CH,), jnp.int32)] * 2
            + [pltpu.SemaphoreType.DMA] * 4
        ),
    )
    def scat_k(x_hbm, idx_hbm, out_hbm, acc, buf0, buf1, ib0, ib1,
               semD0, semD1, semI0, semI1):
        buf = (buf0, buf1)
        ib = (ib0, ib1)
        semD = (semD0, semD1)
        semI = (semI0, semI1)
        cid = lax.axis_index("c")
        sid = lax.axis_index("s")
        wid = sid * NC + cid

        zv = jnp.zeros((16,), F32)

        def zrow(r, carry):
            for q in range(HID // 16):
                buf0[r, pl.ds(q * 16, 16)] = zv
            return carry

        lax.fori_loop(0, CH, zrow, 0)

        def zbody(j, carry):
            c = j * NS + sid

            @pl.when(c < nb_chunks)
            def _():
                pltpu.sync_copy(buf0, acc.at[pl.ds(c * CH, CH)])
            return carry

        lax.fori_loop(0, zit, zbody, 0)
        plsc.subcore_barrier()

        def issue(j, p):
            base = (j * NW + wid) * CH
            pltpu.async_copy(idx_hbm.at[pl.ds(base, CH)], ib[p], semI[p])
            pltpu.async_copy(x_hbm.at[pl.ds(base, CH)], buf[p], semD[p])

        for t in range(2):
            @pl.when(t * NW + wid < n_chunks)
            def _():
                issue(t, t)

        def body(jj, carry):
            for p in range(2):
                j = jj * 2 + p
                c = j * NW + wid

                @pl.when(c < n_chunks)
                def _():
                    pltpu.make_async_copy(
                        idx_hbm.at[pl.ds(0, CH)], ib[p], semI[p]).wait()
                    pltpu.make_async_copy(
                        x_hbm.at[pl.ds(0, CH)], buf[p], semD[p]).wait()
                    pltpu.sync_copy(buf[p], acc.at[ib[p]], add=True)

                    @pl.when(c + 2 * NW < n_chunks)
                    def _():
                        issue(j + 2, p)
            return carry

        lax.fori_loop(0, (iters + 1) // 2, body, 0)
        plsc.subcore_barrier()

        def obody(j, carry):
            c = j * NS + sid

            @pl.when(c < nb_chunks)
            def _():
                b = c * CH
                pltpu.sync_copy(acc.at[pl.ds(b, CH)], buf0)
                pltpu.sync_copy(buf0, out_hbm.at[cid, pl.ds(b, CH)])
            return carry

        lax.fori_loop(0, zit, obody, 0)

    return scat_k(X, idx)


def _sc_dist(zx, zy, zz, src, dst):
    NB = zx.shape[0]
    E = src.shape[0]
    n_chunks = E // CH
    iters = n_chunks // NW

    @functools.partial(
        pl.kernel,
        out_type=jax.ShapeDtypeStruct((E,), F32),
        mesh=_mesh(),
        compiler_params=_sc_params(),
        scratch_types=[
            pltpu.VMEM((NB,), F32),
            pltpu.VMEM((NB,), F32),
            pltpu.VMEM((NB,), F32),
            pltpu.VMEM((CH,), jnp.int32),
            pltpu.VMEM((CH,), jnp.int32),
            pltpu.VMEM((CH,), F32),
        ],
    )
    def dist_k(zx_hbm, zy_hbm, zz_hbm, src_hbm, dst_hbm, d2_hbm,
               xt, yt, zt, sbuf, dbuf, obuf):
        wid = _wid()
        pltpu.sync_copy(zx_hbm, xt)
        pltpu.sync_copy(zy_hbm, yt)
        pltpu.sync_copy(zz_hbm, zt)

        def body(j, carry):
            base = (j * NW + wid) * CH
            pltpu.sync_copy(src_hbm.at[pl.ds(base, CH)], sbuf)
            pltpu.sync_copy(dst_hbm.at[pl.ds(base, CH)], dbuf)
            for g in range(CH // 16):
                si = sbuf[pl.ds(g * 16, 16)]
                di = dbuf[pl.ds(g * 16, 16)]
                rx = plsc.load_gather(xt, [si]) - plsc.load_gather(xt, [di])
                ry = plsc.load_gather(yt, [si]) - plsc.load_gather(yt, [di])
                rz = plsc.load_gather(zt, [si]) - plsc.load_gather(zt, [di])
                obuf[pl.ds(g * 16, 16)] = rx * rx + ry * ry + rz * rz + 1e-8
            pltpu.sync_copy(obuf, d2_hbm.at[pl.ds(base, CH)])
            return carry

        lax.fori_loop(0, iters, body, 0)

    return dist_k(zx, zy, zz, src, dst)


def _sc_msg(A, Bt, C, src, dst):
    NB, HID = A.shape
    E = src.shape[0]
    CHM = 40
    n_chunks = E // CHM
    iters = n_chunks // NW
    QV = HID // 16
    nb_chunks = NB // CHM
    zit = (nb_chunks + NS - 1) // NS

    @functools.partial(
        pl.kernel,
        out_type=jax.ShapeDtypeStruct((NC, NB, HID), F32),
        mesh=_mesh(),
        compiler_params=_sc_params(),
        scratch_types=(
            [pltpu.VMEM_SHARED((NB, HID), F32)]
            + [pltpu.VMEM((CHM,), jnp.int32)] * 8
            + [pltpu.VMEM((CHM, HID), F32)] * 8
            + [pltpu.SemaphoreType.DMA] * 10
        ),
    )
    def msg_k(a_hbm, b_hbm, c_hbm, src_hbm, dst_hbm, agg_hbm,
              acc, sb0, sb1, sb2, sb3, db0, db1, db2, db3,
              ga0, ga1, gb0, gb1, cb0, cb1, mb0, mb1,
              semA0, semB0, semC0, semA1, semB1, semC1,
              semI0, semI1, semI2, semI3):
        sb = (sb0, sb1, sb2, sb3)
        db = (db0, db1, db2, db3)
        ga = (ga0, ga1)
        gb = (gb0, gb1)
        cb = (cb0, cb1)
        mb = (mb0, mb1)
        semA = (semA0, semA1)
        semB = (semB0, semB1)
        semC = (semC0, semC1)
        semI = (semI0, semI1, semI2, semI3)
        cid = lax.axis_index("c")
        sid = lax.axis_index("s")
        wid = sid * NC + cid

        zv = jnp.zeros((16,), F32)

        def zrow(r, carry):
            for q in range(QV):
                mb[0][r, pl.ds(q * 16, 16)] = zv
            return carry

        lax.fori_loop(0, CHM, zrow, 0)

        def zbody(j, carry):
            c = j * NS + sid

            @pl.when(c < nb_chunks)
            def _():
                pltpu.sync_copy(mb[0], acc.at[pl.ds(c * CHM, CHM)])
            return carry

        lax.fori_loop(0, zit, zbody, 0)
        plsc.subcore_barrier()

        def ebase(j):
            return (j * NW + wid) * CHM

        def issue_idx(j, q):
            base = ebase(j)
            pltpu.async_copy(src_hbm.at[pl.ds(base, CHM)], sb[q], semI[q])
            pltpu.async_copy(dst_hbm.at[pl.ds(base, CHM)], db[q], semI[q])

        def wait_idx(q):
            pltpu.make_async_copy(
                src_hbm.at[pl.ds(0, CHM)], sb[q], semI[q]).wait()
            pltpu.make_async_copy(
                dst_hbm.at[pl.ds(0, CHM)], db[q], semI[q]).wait()

        def issue_gath(j, q, p):
            base = ebase(j)
            pltpu.async_copy(a_hbm.at[sb[q]], ga[p], semA[p])
            pltpu.async_copy(b_hbm.at[db[q]], gb[p], semB[p])
            pltpu.async_copy(c_hbm.at[pl.ds(base, CHM)], cb[p], semC[p])

        def wait_gath(q, p):
            pltpu.make_async_copy(a_hbm.at[sb[q]], ga[p], semA[p]).wait()
            pltpu.make_async_copy(b_hbm.at[db[q]], gb[p], semB[p]).wait()
            pltpu.make_async_copy(
                c_hbm.at[pl.ds(0, CHM)], cb[p], semC[p]).wait()

        for t in range(4):
            issue_idx(t, t)
        for t in range(2):
            wait_idx(t)
            issue_gath(t, t, t)

        def body(jj, carry):
            for p4 in range(4):
                p = p4 % 2
                j4 = jj * 4 + p4

                @pl.when(j4 < iters)
                def _():
                    wait_gath(p4, p)

                    def row(r, rc):
                        for q in range(QV):
                            x = (ga[p][r, pl.ds(q * 16, 16)]
                                 + gb[p][r, pl.ds(q * 16, 16)]
                                 + cb[p][r, pl.ds(q * 16, 16)])
                            mb[p][r, pl.ds(q * 16, 16)] = (
                                x / (1.0 + jnp.exp(-x)))
                        return rc

                    lax.fori_loop(0, CHM, row, 0)
                    pltpu.sync_copy(mb[p], acc.at[db[p4]], add=True)

                    @pl.when(j4 + 2 < iters)
                    def _():
                        wait_idx((p4 + 2) % 4)
                        issue_gath(j4 + 2, (p4 + 2) % 4, p)

                    @pl.when(j4 + 4 < iters)
                    def _():
                        issue_idx(j4 + 4, p4)
            return carry

        lax.fori_loop(0, (iters + 3) // 4, body, 0)
        plsc.subcore_barrier()

        def obody(j, carry):
            c = j * NS + sid

            @pl.when(c < nb_chunks)
            def _():
                b = c * CHM
                pltpu.sync_copy(acc.at[pl.ds(b, CHM)], ga0)
                pltpu.sync_copy(ga0, agg_hbm.at[cid, pl.ds(b, CHM)])
            return carry

        lax.fori_loop(0, zit, obody, 0)

    return msg_k(A, Bt, C, src, dst)


def _tc_combine(hsum, zsum, WmA, WmB):
    _, NB, HID = hsum.shape
    BK = 1000
    grid = NB // BK

    def body(hs_ref, zs_ref, wa_ref, wb_ref, hb_ref, zb_ref, a_ref, b_ref):
        hs = hs_ref[0] + hs_ref[1]
        zs = zs_ref[0] + zs_ref[1]
        cnt = zs[:, 3:4]
        inv = 1.0 / jnp.maximum(cnt, 1.0)
        hb = hs * inv
        hb_ref[...] = hb
        zb_ref[...] = zs[:, :16] * inv
        a_ref[...] = jnp.dot(hb, wa_ref[...], preferred_element_type=F32)
        b_ref[...] = jnp.dot(hb, wb_ref[...], preferred_element_type=F32)

    return pl.pallas_call(
        body,
        grid=(grid,),
        in_specs=[
            pl.BlockSpec((2, BK, HID), lambda i: (0, i, 0)),
            pl.BlockSpec((2, BK, HID), lambda i: (0, i, 0)),
            pl.BlockSpec((HID, HID), lambda i: (0, 0)),
            pl.BlockSpec((HID, HID), lambda i: (0, 0)),
        ],
        out_specs=[
            pl.BlockSpec((BK, HID), lambda i: (i, 0)),
            pl.BlockSpec((BK, 16), lambda i: (i, 0)),
            pl.BlockSpec((BK, HID), lambda i: (i, 0)),
            pl.BlockSpec((BK, HID), lambda i: (i, 0)),
        ],
        out_shape=[
            jax.ShapeDtypeStruct((NB, HID), F32),
            jax.ShapeDtypeStruct((NB, 16), F32),
            jax.ShapeDtypeStruct((NB, HID), F32),
            jax.ShapeDtypeStruct((NB, HID), F32),
        ],
    )(hsum, zsum, WmA, WmB)


def _tc_edge(edge_attr, d2, Wes, wds, bms):
    E, EDGE = edge_attr.shape
    HID = Wes[0].shape[1]
    BK = 2000
    grid = E // BK

    def body(ea_ref, d2_ref, we0, we1, we2, wd0, wd1, wd2, b0, b1, b2,
             c0_ref, c1_ref, c2_ref):
        ea = ea_ref[...]
        dist = jnp.sqrt(d2_ref[...])
        for w, wd, b, ref in ((we0, wd0, b0, c0_ref),
                              (we1, wd1, b1, c1_ref),
                              (we2, wd2, b2, c2_ref)):
            ref[...] = (jnp.dot(ea, w[...], preferred_element_type=F32)
                        + dist * wd[...] + b[...])

    wspec = pl.BlockSpec((EDGE, HID), lambda i: (0, 0))
    rspec = pl.BlockSpec((1, HID), lambda i: (0, 0))
    espec = pl.BlockSpec((BK, HID), lambda i: (i, 0))
    return pl.pallas_call(
        body,
        grid=(grid,),
        in_specs=[pl.BlockSpec((BK, EDGE), lambda i: (i, 0)),
                  pl.BlockSpec((BK, 1), lambda i: (i, 0)),
                  wspec, wspec, wspec, rspec, rspec, rspec,
                  rspec, rspec, rspec],
        out_specs=[espec, espec, espec],
        out_shape=[jax.ShapeDtypeStruct((E, HID), F32)] * 3,
    )(edge_attr, d2, Wes[0], Wes[1], Wes[2], wds[0], wds[1], wds[2],
      bms[0], bms[1], bms[2])


def _tc_update(h, aggp, WuH, WuA, bu, WmA, WmB):
    NB, HID = h.shape
    BK = 1000
    grid = NB // BK
    last = WmA is None

    if last:
        ins = (h, aggp, WuH, WuA, bu)
        n_out = 1
    else:
        ins = (h, aggp, WuH, WuA, bu, WmA, WmB)
        n_out = 3

    wspec = pl.BlockSpec((HID, HID), lambda i: (0, 0))
    hspec = pl.BlockSpec((BK, HID), lambda i: (i, 0))

    def body2(*refs):
        if last:
            h_ref, ag_ref, wh_ref, wa_ref, bu_ref, hn_ref = refs
        else:
            (h_ref, ag_ref, wh_ref, wa_ref, bu_ref, wma_ref, wmb_ref,
             hn_ref, a_ref, b_ref) = refs
        h0 = h_ref[...]
        agg = ag_ref[0] + ag_ref[1]
        pre = (jnp.dot(h0, wh_ref[...], preferred_element_type=F32)
               + jnp.dot(agg, wa_ref[...], preferred_element_type=F32)
               + bu_ref[...])
        hn = h0 + pre * (1.0 / (1.0 + jnp.exp(-pre)))
        hn_ref[...] = hn
        if not last:
            a_ref[...] = jnp.dot(hn, wma_ref[...], preferred_element_type=F32)
            b_ref[...] = jnp.dot(hn, wmb_ref[...], preferred_element_type=F32)

    in_specs = [hspec, pl.BlockSpec((2, BK, HID), lambda i: (0, i, 0)),
                wspec, wspec, pl.BlockSpec((1, HID), lambda i: (0, 0))]
    if not last:
        in_specs += [wspec, wspec]
    return pl.pallas_call(
        body2,
        grid=(grid,),
        in_specs=in_specs,
        out_specs=[hspec] * n_out,
        out_shape=[jax.ShapeDtypeStruct((NB, HID), F32)] * n_out,
    )(*ins)


def _tc_final(h, gmask, batch_row, B):
    NB, HID = h.shape
    BK = 1000
    grid = NB // BK

    def body(h_ref, m_ref, bid_ref, br_ref, gr_ref, gs_ref):
        i = pl.program_id(0)
        h0 = h_ref[...]
        nrm = jnp.sqrt(jnp.sum(h0 * h0, axis=1, keepdims=True))
        br = h0 / jnp.maximum(nrm, 1e-12)
        br_ref[...] = br
        masked = br * m_ref[...]
        bid = bid_ref[0]
        oh = (lax.broadcasted_iota(jnp.int32, (B, BK), 0)
              == jnp.broadcast_to(bid, (B, BK))).astype(F32)
        part = jnp.dot(oh, masked, preferred_element_type=F32)

        @pl.when(i == 0)
        def _():
            gs_ref[...] = jnp.zeros_like(gs_ref)

        gs_ref[...] += part

        @pl.when(i == grid - 1)
        def _():
            gs = gs_ref[...]
            gn = jnp.sqrt(jnp.sum(gs * gs, axis=1, keepdims=True))
            gr_ref[...] = gs / jnp.maximum(gn, 1e-12)

    return pl.pallas_call(
        body,
        grid=(grid,),
        in_specs=[
            pl.BlockSpec((BK, HID), lambda i: (i, 0)),
            pl.BlockSpec((BK, 1), lambda i: (i, 0)),
            pl.BlockSpec((1, 1, BK), lambda i: (i, 0, 0)),
        ],
        out_specs=[
            pl.BlockSpec((BK, HID), lambda i: (i, 0)),
            pl.BlockSpec((B, HID), lambda i: (0, 0)),
        ],
        out_shape=[
            jax.ShapeDtypeStruct((NB, HID), F32),
            jax.ShapeDtypeStruct((B, HID), F32),
        ],
        scratch_shapes=[pltpu.VMEM((B, HID), F32)],
    )(h, gmask, batch_row)


def kernel(H, Z, block_id, batch_id, perturb_mask, edges, edge_attr,
           global_mask, Wm0, bm0, Wu0, bu0, Wm1, bm1, Wu1, bu1,
           Wm2, bm2, Wu2, bu2):
    NA, HID = H.shape
    NB = global_mask.shape[0]
    E = edge_attr.shape[0]
    B = 32

    src = edges[0].astype(jnp.int32)
    dst = edges[1].astype(jnp.int32)
    bid = block_id.astype(jnp.int32)

    Zaug = jnp.concatenate(
        [Z.reshape(NA, 3), jnp.ones((NA, 1), F32),
         jnp.zeros((NA, HID - 4), F32)], axis=1)

    Wm = (Wm0, Wm1, Wm2)
    WmA = [w[:HID] for w in Wm]
    WmB = [w[HID:2 * HID] for w in Wm]
    Wes = [w[2 * HID:2 * HID + 16] for w in Wm]
    wds = [w[2 * HID + 16:] for w in Wm]
    bms = [b.reshape(1, HID) for b in (bm0, bm1, bm2)]
    Wu = (Wu0, Wu1, Wu2)
    WuH = [w[:HID] for w in Wu]
    WuA = [w[HID:] for w in Wu]
    bus = [b.reshape(1, HID) for b in (bu0, bu1, bu2)]

    hsum = _sc_scatter128(H, bid, NB)
    zsum = _sc_scatter128(Zaug, bid, NB)
    Hb, Zb16, A, Bt = _tc_combine(hsum, zsum, WmA[0], WmB[0])

    d2 = _sc_dist(Zb16[:, 0], Zb16[:, 1], Zb16[:, 2], src, dst)
    C0, C1, C2 = _tc_edge(edge_attr, d2.reshape(E, 1), Wes, wds, bms)
    Cs = (C0, C1, C2)

    h = Hb
    for l in range(3):
        aggp = _sc_msg(A, Bt, Cs[l], src, dst)
        if l < 2:
            h, A, Bt = _tc_update(h, aggp, WuH[l], WuA[l], bus[l],
                                  WmA[l + 1], WmB[l + 1])
        else:
            (h,) = _tc_update(h, aggp, WuH[l], WuA[l], bus[l], None, None)

    gm = global_mask.astype(F32).reshape(NB, 1)
    batch_row = batch_id.astype(jnp.int32).reshape(NB // 1000, 1, 1000)
    block_repr, graph_repr = _tc_final(h, gm, batch_row, B)
    return (Hb, block_repr, graph_repr, None)
